# baseline (speedup 1.0000x reference)
import functools

import jax
import jax.numpy as jnp
from jax import lax
from jax.experimental import pallas as pl
from jax.experimental.pallas import tpu as pltpu
from jax.experimental.pallas import tpu_sc as plsc

_NC = 2
_NS = 16
_NW = _NC * _NS
_BT = 128


@functools.partial(jax.jit, static_argnums=(2, 3, 4))
def _sc_gather_t(idxp, w_wide, h, b, d):
  n_units = h * (b // _BT)
  upw = n_units // _NW
  mesh = plsc.VectorSubcoreMesh(
      core_axis_name="c", subcore_axis_name="s", num_cores=_NC,
      num_subcores=_NS)

  @functools.partial(
      pl.kernel,
      out_type=jax.ShapeDtypeStruct((h, d, b), jnp.float32),
      mesh=mesh,
      scratch_types=[
          pltpu.VMEM((upw, _BT), jnp.int32),
          pltpu.VMEM((_BT, 128), jnp.float32),
          pltpu.VMEM((_BT, 128), jnp.float32),
          pltpu.VMEM((d, _BT), jnp.float32),
          pltpu.VMEM((d, _BT), jnp.float32),
          [pltpu.SemaphoreType.DMA] * 2,
          [pltpu.SemaphoreType.DMA] * 2,
      ],
      compiler_params=pltpu.CompilerParams(
          use_tc_tiling_on_sc=True, needs_layout_passes=False),
  )
  def grab(idx_hbm, w_hbm, out_hbm, idx_v, p0, p1, o0, o1, gsems, ssems):
    wid = lax.axis_index("s") * _NC + lax.axis_index("c")
    pltpu.sync_copy(idx_hbm.at[wid], idx_v)
    pbufs = (p0, p1)
    obufs = (o0, o1)
    base_iota = jax.lax.iota(jnp.int32, 16)
    rows16 = [base_iota + i0 * 16 for i0 in range(8)]

    def out_slice(u):
      g = wid * upw + u
      hh = g // (b // _BT)
      bt = g % (b // _BT)
      return out_hbm.at[hh, :, pl.ds(bt * _BT, _BT)]

    for s in range(2):
      pltpu.async_copy(w_hbm.at[idx_v.at[s]], pbufs[s], gsems[s])

    @pl.loop(0, upw // 2)
    def _(up):
      for s in range(2):
        u = up * 2 + s
        pltpu.make_async_copy(
            w_hbm.at[idx_v.at[u]], pbufs[s], gsems[s]).wait()

        @pl.when(u >= 2)
        def _():
          pltpu.make_async_copy(obufs[s], out_slice(u - 2), ssems[s]).wait()

        for dd in range(d):
          col16 = jnp.full((16,), dd, jnp.int32)
          for i0 in range(8):
            v16 = plsc.load_gather(pbufs[s], [rows16[i0], col16])
            obufs[s][dd, pl.ds(i0 * 16, 16)] = v16

        pltpu.async_copy(obufs[s], out_slice(u), ssems[s])

        @pl.when(u + 2 < upw)
        def _():
          pltpu.async_copy(w_hbm.at[idx_v.at[u + 2]], pbufs[s], gsems[s])

    for s in range(2):
      u = upw - 2 + s
      pltpu.make_async_copy(obufs[s], out_slice(u), ssems[s]).wait()

  return grab(idxp, w_wide)


def kernel(input_, weight):
  b, h = input_.shape
  v, d = weight.shape
  n_units = h * (b // _BT)
  idxp = input_.T.reshape(_NW, n_units // _NW, _BT)
  w_wide = jnp.pad(weight, ((0, 0), (0, 128 - d)))
  out_t = _sc_gather_t(idxp, w_wide, h, b, d)
  return out_t.transpose(2, 0, 1)


# speedup vs baseline: 2.5622x; 2.5622x over previous
import functools

import jax
import jax.numpy as jnp
from jax import lax
from jax.experimental import pallas as pl
from jax.experimental.pallas import tpu as pltpu
from jax.experimental.pallas import tpu_sc as plsc

_NC = 2
_NS = 16
_NW = _NC * _NS
_BT = 128


@functools.partial(jax.jit, static_argnums=(2, 3, 4))
def _sc_gather_t(idxp, w_wide, h, b, d):
  n_units = h * (b // _BT)
  upw = n_units // _NW
  mesh = plsc.VectorSubcoreMesh(
      core_axis_name="c", subcore_axis_name="s", num_cores=_NC,
      num_subcores=_NS)

  @functools.partial(
      pl.kernel,
      out_type=jax.ShapeDtypeStruct((h, d, b), jnp.float32),
      mesh=mesh,
      scratch_types=[
          pltpu.VMEM((upw, _BT), jnp.int32),
          pltpu.VMEM((_BT, 128), jnp.float32),
          pltpu.VMEM((_BT, 128), jnp.float32),
          pltpu.VMEM((d, _BT), jnp.float32),
          pltpu.VMEM((d, _BT), jnp.float32),
          [pltpu.SemaphoreType.DMA] * 2,
          [pltpu.SemaphoreType.DMA] * 2,
      ],
      compiler_params=pltpu.CompilerParams(
          use_tc_tiling_on_sc=True, needs_layout_passes=False),
  )
  def grab(idx_hbm, w_hbm, out_hbm, idx_v, p0, p1, o0, o1, gsems, ssems):
    wid = lax.axis_index("s") * _NC + lax.axis_index("c")
    pltpu.sync_copy(idx_hbm.at[wid], idx_v)
    pbufs = (p0, p1)
    obufs = (o0, o1)
    base_iota = jax.lax.iota(jnp.int32, 16)
    rows16 = [base_iota + i0 * 16 for i0 in range(8)]

    def out_slice(u):
      g = wid * upw + u
      hh = g // (b // _BT)
      bt = g % (b // _BT)
      return out_hbm.at[hh, :, pl.ds(bt * _BT, _BT)]

    for s in range(2):
      pltpu.async_copy(w_hbm.at[idx_v.at[s]], pbufs[s], gsems[s])

    @pl.loop(0, upw // 2)
    def _(up):
      for s in range(2):
        u = up * 2 + s
        pltpu.make_async_copy(
            w_hbm.at[idx_v.at[u]], pbufs[s], gsems[s]).wait()

        @pl.when(u >= 2)
        def _():
          pltpu.make_async_copy(obufs[s], out_slice(u - 2), ssems[s]).wait()

        @functools.partial(plsc.parallel_loop, 0, d, unroll=4)
        def _(dd):
          col16 = jnp.full((16,), dd, jnp.int32)
          for i0 in range(8):
            v16 = plsc.load_gather(pbufs[s], [rows16[i0], col16])
            obufs[s][dd, pl.ds(i0 * 16, 16)] = v16

        pltpu.async_copy(obufs[s], out_slice(u), ssems[s])

        @pl.when(u + 2 < upw)
        def _():
          pltpu.async_copy(w_hbm.at[idx_v.at[u + 2]], pbufs[s], gsems[s])

    for s in range(2):
      u = upw - 2 + s
      pltpu.make_async_copy(obufs[s], out_slice(u), ssems[s]).wait()

  return grab(idxp, w_wide)


def kernel(input_, weight):
  b, h = input_.shape
  v, d = weight.shape
  n_units = h * (b // _BT)
  idxp = input_.T.reshape(_NW, n_units // _NW, _BT)
  w_wide = jnp.pad(weight, ((0, 0), (0, 128 - d)))
  out_t = _sc_gather_t(idxp, w_wide, h, b, d)
  return out_t.transpose(2, 0, 1)
